# baseline (speedup 1.0000x reference)
import jax
import jax.numpy as jnp
from jax.experimental import pallas as pl
from jax.experimental.pallas import tpu as pltpu

HIDDEN = 128
NUM_CLASSES = 10
C_PAD = 128
B_TILE = 512
N_CHUNKS = 2


def _gates_of(g, H):
    i = jnp.tanh(g[:, 0 * H:1 * H])
    f = jnp.tanh(g[:, 1 * H:2 * H])
    gg = jnp.tanh(g[:, 2 * H:3 * H])
    o = jnp.tanh(g[:, 3 * H:4 * H])
    return i, f, gg, o


def _cell(i, f, gg, o, c):
    c_new = 0.5 * ((f * c + c) + (i * gg + gg))
    t = jnp.tanh(c_new)
    return c_new, o * t + t


def _make_body(first, last):
    def body(*refs):
        if first:
            x_ref, w1_ref, wp_ref = refs[:3]
            rest = refs[3:]
        else:
            x_ref, w1_ref, wp_ref, sin_ref = refs[:4]
            rest = refs[4:]
        if last:
            out_ref, = rest
        else:
            sout_ref, = rest

        Tc, Bt, _ = x_ref.shape
        H = w1_ref.shape[1] // 4

        w1 = w1_ref[...]
        b1 = wp_ref[256:257, :]
        w2 = wp_ref[0:256, :]
        b2 = wp_ref[257:258, :]

        if first:
            h1 = jnp.zeros((Bt, H), jnp.float32)
            c1 = jnp.zeros((Bt, H), jnp.float32)
            h2 = jnp.zeros((Bt, H), jnp.float32)
            c2 = jnp.zeros((Bt, H), jnp.float32)
        else:
            h1 = sin_ref[0]
            c1 = sin_ref[1]
            h2 = sin_ref[2]
            c2 = sin_ref[3]

        for t in range(Tc):
            lhs1 = jnp.concatenate([h1.astype(x_ref.dtype), x_ref[t]], axis=1)
            g1 = jnp.dot(lhs1, w1, preferred_element_type=jnp.float32) + b1
            i1, f1, gg1, o1 = _gates_of(g1, H)
            c1, h1 = _cell(i1, f1, gg1, o1, c1)

            g2 = jnp.dot(jnp.concatenate([h1, h2], axis=1), w2,
                         preferred_element_type=jnp.float32) + b2
            i2, f2, gg2, o2 = _gates_of(g2, H)
            c2, h2 = _cell(i2, f2, gg2, o2, c2)

        if last:
            wfc = wp_ref[258:386, 0:C_PAD]
            bfc = wp_ref[386:387, 0:C_PAD]
            out_ref[...] = (jnp.dot(h2, wfc, preferred_element_type=jnp.float32)
                            + bfc)
        else:
            sout_ref[0] = h1
            sout_ref[1] = c1
            sout_ref[2] = h2
            sout_ref[3] = c2

    return body


def kernel(x, w_ih_l0, w_hh_l0, b_l0, w_ih_l1, w_hh_l1, b_l1, w_fc, b_fc):
    B, T, I = x.shape
    H = w_hh_l0.shape[1]
    C = w_fc.shape[0]

    gate_scale = jnp.concatenate([
        jnp.full((2 * H,), 0.5, jnp.float32),
        jnp.ones((H,), jnp.float32),
        jnp.full((H,), 0.5, jnp.float32),
    ])[None, :]
    w1 = (jnp.concatenate([0.5 * w_hh_l0.T, w_ih_l0.T], axis=0)
          * gate_scale).astype(jnp.bfloat16)
    w2p = jnp.concatenate([0.5 * w_ih_l1.T, 0.5 * w_hh_l1.T], axis=0) * gate_scale
    b1p = b_l0[None, :] * gate_scale
    b2p = b_l1[None, :] * gate_scale
    wfcp = jnp.zeros((H, 4 * H), jnp.float32).at[:H, :C].set(0.5 * w_fc.T)
    bfcp = jnp.zeros((1, 4 * H), jnp.float32).at[:, :C].set(b_fc[None, :])
    wpack = jnp.concatenate([w2p, b1p, b2p, wfcp, bfcp], axis=0)

    n_btiles = B // B_TILE
    const = lambda b: (0, 0)
    Tc = T // N_CHUNKS

    x_chunks = [
        jnp.transpose(x[:, k * Tc:(k + 1) * Tc, :], (1, 0, 2)).astype(jnp.bfloat16)
        for k in range(N_CHUNKS)
    ]

    x_spec = pl.BlockSpec((Tc, B_TILE, I), lambda b: (0, b, 0))
    w1_spec = pl.BlockSpec((H + I, 4 * H), const)
    wp_spec = pl.BlockSpec((387, 4 * H), const)
    state_spec = pl.BlockSpec((4, B_TILE, H), lambda b: (0, b, 0))
    cparams = pltpu.CompilerParams(dimension_semantics=("parallel",))

    res = None
    state = None
    for k in range(N_CHUNKS):
        first = k == 0
        last = k == N_CHUNKS - 1
        in_specs = [x_spec, w1_spec, wp_spec] + ([] if first else [state_spec])
        args = [x_chunks[k], w1, wpack] + ([] if first else [state])
        if last:
            out_shape = jax.ShapeDtypeStruct((B, C_PAD), jnp.float32)
            out_spec = pl.BlockSpec((B_TILE, C_PAD), lambda b: (b, 0))
        else:
            out_shape = jax.ShapeDtypeStruct((4, B, H), jnp.float32)
            out_spec = state_spec
        res = pl.pallas_call(
            _make_body(first, last),
            out_shape=out_shape,
            grid=(n_btiles,),
            in_specs=in_specs,
            out_specs=out_spec,
            compiler_params=cparams,
        )(*args)
        state = res

    return res[:, :C]


# speedup vs baseline: 1.1642x; 1.1642x over previous
import jax
import jax.numpy as jnp
from jax.experimental import pallas as pl
from jax.experimental.pallas import tpu as pltpu

HIDDEN = 128
NUM_CLASSES = 10
C_PAD = 128
B_TILE = 512
N_CHUNKS = 1


def _gates_of(g, H):
    i = jnp.tanh(g[:, 0 * H:1 * H])
    f = jnp.tanh(g[:, 1 * H:2 * H])
    gg = jnp.tanh(g[:, 2 * H:3 * H])
    o = jnp.tanh(g[:, 3 * H:4 * H])
    return i, f, gg, o


def _cell(i, f, gg, o, c):
    c_new = 0.5 * ((f * c + c) + (i * gg + gg))
    t = jnp.tanh(c_new)
    return c_new, o * t + t


def _make_body(first, last):
    def body(*refs):
        if first:
            x_ref, w1_ref, wp_ref = refs[:3]
            rest = refs[3:]
        else:
            x_ref, w1_ref, wp_ref, sin_ref = refs[:4]
            rest = refs[4:]
        if last:
            out_ref, = rest
        else:
            sout_ref, = rest

        Tc, Bt, _ = x_ref.shape
        H = w1_ref.shape[1] // 4

        w1 = w1_ref[...]
        b1 = wp_ref[256:257, :]
        w2 = wp_ref[0:256, :]
        b2 = wp_ref[257:258, :]

        if first:
            h1 = jnp.zeros((Bt, H), jnp.float32)
            c1 = jnp.zeros((Bt, H), jnp.float32)
            h2 = jnp.zeros((Bt, H), jnp.float32)
            c2 = jnp.zeros((Bt, H), jnp.float32)
        else:
            h1 = sin_ref[0]
            c1 = sin_ref[1]
            h2 = sin_ref[2]
            c2 = sin_ref[3]

        for t in range(Tc):
            lhs1 = jnp.concatenate([h1.astype(x_ref.dtype), x_ref[t]], axis=1)
            g1 = jnp.dot(lhs1, w1, preferred_element_type=jnp.float32) + b1
            i1, f1, gg1, o1 = _gates_of(g1, H)
            c1, h1 = _cell(i1, f1, gg1, o1, c1)

            g2 = jnp.dot(jnp.concatenate([h1, h2], axis=1), w2,
                         preferred_element_type=jnp.float32) + b2
            i2, f2, gg2, o2 = _gates_of(g2, H)
            c2, h2 = _cell(i2, f2, gg2, o2, c2)

        if last:
            wfc = wp_ref[258:386, 0:C_PAD]
            bfc = wp_ref[386:387, 0:C_PAD]
            out_ref[...] = (jnp.dot(h2, wfc, preferred_element_type=jnp.float32)
                            + bfc)
        else:
            sout_ref[0] = h1
            sout_ref[1] = c1
            sout_ref[2] = h2
            sout_ref[3] = c2

    return body


def kernel(x, w_ih_l0, w_hh_l0, b_l0, w_ih_l1, w_hh_l1, b_l1, w_fc, b_fc):
    B, T, I = x.shape
    H = w_hh_l0.shape[1]
    C = w_fc.shape[0]

    gate_scale = jnp.concatenate([
        jnp.full((2 * H,), 0.5, jnp.float32),
        jnp.ones((H,), jnp.float32),
        jnp.full((H,), 0.5, jnp.float32),
    ])[None, :]
    w1 = (jnp.concatenate([0.5 * w_hh_l0.T, w_ih_l0.T], axis=0)
          * gate_scale).astype(jnp.bfloat16)
    w2p = jnp.concatenate([0.5 * w_ih_l1.T, 0.5 * w_hh_l1.T], axis=0) * gate_scale
    b1p = b_l0[None, :] * gate_scale
    b2p = b_l1[None, :] * gate_scale
    wfcp = jnp.zeros((H, 4 * H), jnp.float32).at[:H, :C].set(0.5 * w_fc.T)
    bfcp = jnp.zeros((1, 4 * H), jnp.float32).at[:, :C].set(b_fc[None, :])
    wpack = jnp.concatenate([w2p, b1p, b2p, wfcp, bfcp], axis=0)

    n_btiles = B // B_TILE
    const = lambda b: (0, 0)
    Tc = T // N_CHUNKS

    x_chunks = [
        jnp.transpose(x[:, k * Tc:(k + 1) * Tc, :], (1, 0, 2)).astype(jnp.bfloat16)
        for k in range(N_CHUNKS)
    ]

    x_spec = pl.BlockSpec((Tc, B_TILE, I), lambda b: (0, b, 0))
    w1_spec = pl.BlockSpec((H + I, 4 * H), const)
    wp_spec = pl.BlockSpec((387, 4 * H), const)
    state_spec = pl.BlockSpec((4, B_TILE, H), lambda b: (0, b, 0))
    cparams = pltpu.CompilerParams(dimension_semantics=("parallel",))

    res = None
    state = None
    for k in range(N_CHUNKS):
        first = k == 0
        last = k == N_CHUNKS - 1
        in_specs = [x_spec, w1_spec, wp_spec] + ([] if first else [state_spec])
        args = [x_chunks[k], w1, wpack] + ([] if first else [state])
        if last:
            out_shape = jax.ShapeDtypeStruct((B, C_PAD), jnp.float32)
            out_spec = pl.BlockSpec((B_TILE, C_PAD), lambda b: (b, 0))
        else:
            out_shape = jax.ShapeDtypeStruct((4, B, H), jnp.float32)
            out_spec = state_spec
        res = pl.pallas_call(
            _make_body(first, last),
            out_shape=out_shape,
            grid=(n_btiles,),
            in_specs=in_specs,
            out_specs=out_spec,
            compiler_params=cparams,
        )(*args)
        state = res

    return res[:, :C]


# speedup vs baseline: 1.1650x; 1.0007x over previous
import jax
import jax.numpy as jnp
from jax.experimental import pallas as pl
from jax.experimental.pallas import tpu as pltpu

HIDDEN = 128
NUM_CLASSES = 10
C_PAD = 128
B_TILE = 512
N_CHUNKS = 1


def _gates_of(g, H):
    i = jnp.tanh(g[:, 0 * H:1 * H])
    f = jnp.tanh(g[:, 1 * H:2 * H])
    gg = jnp.tanh(g[:, 2 * H:3 * H])
    o = jnp.tanh(g[:, 3 * H:4 * H])
    return i, f, gg, o


def _cell(i, f, gg, o, c):
    c_new = 0.5 * ((f * c + c) + (i * gg + gg))
    t = jnp.tanh(c_new)
    return c_new, o * t + t


def _make_body(first, last):
    def body(*refs):
        if first:
            x_ref, w1_ref, wp_ref = refs[:3]
            rest = refs[3:]
        else:
            x_ref, w1_ref, wp_ref, sin_ref = refs[:4]
            rest = refs[4:]
        if last:
            out_ref, = rest
        else:
            sout_ref, = rest

        Tc, Bt, _ = x_ref.shape
        H = w1_ref.shape[1] // 4

        w1 = w1_ref[...]
        b1 = wp_ref[256:257, :]
        w2 = wp_ref[0:256, :]
        b2 = wp_ref[257:258, :]

        if first:
            h1 = jnp.zeros((Bt, H), jnp.float32)
            c1 = jnp.zeros((Bt, H), jnp.float32)
            h2 = jnp.zeros((Bt, H), jnp.float32)
            c2 = jnp.zeros((Bt, H), jnp.float32)
        else:
            h1 = sin_ref[0]
            c1 = sin_ref[1]
            h2 = sin_ref[2]
            c2 = sin_ref[3]

        for t in range(Tc):
            lhs1 = jnp.concatenate([h1.astype(x_ref.dtype), x_ref[t]], axis=1)
            g1 = jnp.dot(lhs1, w1, preferred_element_type=jnp.float32) + b1
            i1, f1, gg1, o1 = _gates_of(g1, H)
            c1, h1 = _cell(i1, f1, gg1, o1, c1)

            g2 = jnp.dot(jnp.concatenate([h1, h2], axis=1), w2,
                         preferred_element_type=jnp.float32) + b2
            i2, f2, gg2, o2 = _gates_of(g2, H)
            c2, h2 = _cell(i2, f2, gg2, o2, c2)

        if last:
            wfc = wp_ref[258:386, 0:C_PAD]
            bfc = wp_ref[386:387, 0:C_PAD]
            out_ref[...] = (jnp.dot(h2, wfc, preferred_element_type=jnp.float32)
                            + bfc)
        else:
            sout_ref[0] = h1
            sout_ref[1] = c1
            sout_ref[2] = h2
            sout_ref[3] = c2

    return body


def kernel(x, w_ih_l0, w_hh_l0, b_l0, w_ih_l1, w_hh_l1, b_l1, w_fc, b_fc):
    B, T, I = x.shape
    H = w_hh_l0.shape[1]
    C = w_fc.shape[0]

    gate_scale = jnp.concatenate([
        jnp.full((2 * H,), 0.5, jnp.float32),
        jnp.ones((H,), jnp.float32),
        jnp.full((H,), 0.5, jnp.float32),
    ])[None, :]
    w1 = (jnp.concatenate([0.5 * w_hh_l0.T, w_ih_l0.T], axis=0)
          * gate_scale).astype(jnp.bfloat16)
    w2p = jnp.concatenate([0.5 * w_ih_l1.T, 0.5 * w_hh_l1.T], axis=0) * gate_scale
    b1p = b_l0[None, :] * gate_scale
    b2p = b_l1[None, :] * gate_scale
    wfcp = jnp.zeros((H, 4 * H), jnp.float32).at[:H, :C].set(0.5 * w_fc.T)
    bfcp = jnp.zeros((1, 4 * H), jnp.float32).at[:, :C].set(b_fc[None, :])
    wpack = jnp.concatenate([w2p, b1p, b2p, wfcp, bfcp], axis=0)

    n_btiles = B // B_TILE
    const = lambda b: (0, 0)
    Tc = T // N_CHUNKS

    x_chunks = [
        jnp.transpose(x[:, k * Tc:(k + 1) * Tc, :].astype(jnp.bfloat16), (1, 0, 2))
        for k in range(N_CHUNKS)
    ]

    x_spec = pl.BlockSpec((Tc, B_TILE, I), lambda b: (0, b, 0))
    w1_spec = pl.BlockSpec((H + I, 4 * H), const)
    wp_spec = pl.BlockSpec((387, 4 * H), const)
    state_spec = pl.BlockSpec((4, B_TILE, H), lambda b: (0, b, 0))
    cparams = pltpu.CompilerParams(dimension_semantics=("parallel",))

    res = None
    state = None
    for k in range(N_CHUNKS):
        first = k == 0
        last = k == N_CHUNKS - 1
        in_specs = [x_spec, w1_spec, wp_spec] + ([] if first else [state_spec])
        args = [x_chunks[k], w1, wpack] + ([] if first else [state])
        if last:
            out_shape = jax.ShapeDtypeStruct((B, C_PAD), jnp.float32)
            out_spec = pl.BlockSpec((B_TILE, C_PAD), lambda b: (b, 0))
        else:
            out_shape = jax.ShapeDtypeStruct((4, B, H), jnp.float32)
            out_spec = state_spec
        res = pl.pallas_call(
            _make_body(first, last),
            out_shape=out_shape,
            grid=(n_btiles,),
            in_specs=in_specs,
            out_specs=out_spec,
            compiler_params=cparams,
        )(*args)
        state = res

    return res[:, :C]


# speedup vs baseline: 1.1742x; 1.0079x over previous
import jax
import jax.numpy as jnp
from jax.experimental import pallas as pl
from jax.experimental.pallas import tpu as pltpu

HIDDEN = 128
NUM_CLASSES = 10
C_PAD = 128
B_TILE = 512
N_CHUNKS = 1


def _gates_of(g, H):
    i = jnp.tanh(g[:, 0 * H:1 * H])
    f = jnp.tanh(g[:, 1 * H:2 * H])
    gg = jnp.tanh(g[:, 2 * H:3 * H])
    o = jnp.tanh(g[:, 3 * H:4 * H])
    return i, f, gg, o


def _cell(i, f, gg, o, c):
    c_new = 0.5 * ((f * c + c) + (i * gg + gg))
    t = jnp.tanh(c_new)
    return c_new, o * t + t


def _make_body(first, last):
    def body(*refs):
        if first:
            x_ref, w1_ref, wp_ref = refs[:3]
            rest = refs[3:]
        else:
            x_ref, w1_ref, wp_ref, sin_ref = refs[:4]
            rest = refs[4:]
        if last:
            out_ref, = rest
        else:
            sout_ref, = rest

        Tc, Bt, _ = x_ref.shape
        H = w1_ref.shape[1] // 4

        w1 = w1_ref[...]
        w2 = wp_ref[0:256, :]
        b2 = wp_ref[256:257, :]

        if first:
            h1 = jnp.zeros((Bt, H), jnp.float32)
            c1 = jnp.zeros((Bt, H), jnp.float32)
            h2 = jnp.zeros((Bt, H), jnp.float32)
            c2 = jnp.zeros((Bt, H), jnp.float32)
        else:
            h1 = sin_ref[0]
            c1 = sin_ref[1]
            h2 = sin_ref[2]
            c2 = sin_ref[3]

        ones_col = jnp.ones((Bt, 1), x_ref.dtype)

        for t in range(Tc):
            lhs1 = jnp.concatenate(
                [h1.astype(x_ref.dtype), x_ref[t], ones_col], axis=1)
            g1 = jnp.dot(lhs1, w1, preferred_element_type=jnp.float32)
            i1, f1, gg1, o1 = _gates_of(g1, H)
            c1, h1 = _cell(i1, f1, gg1, o1, c1)

            g2 = jnp.dot(jnp.concatenate([h1, h2], axis=1), w2,
                         preferred_element_type=jnp.float32) + b2
            i2, f2, gg2, o2 = _gates_of(g2, H)
            c2, h2 = _cell(i2, f2, gg2, o2, c2)

        if last:
            wfc = wp_ref[257:385, 0:C_PAD]
            bfc = wp_ref[385:386, 0:C_PAD]
            out_ref[...] = (jnp.dot(h2, wfc, preferred_element_type=jnp.float32)
                            + bfc)
        else:
            sout_ref[0] = h1
            sout_ref[1] = c1
            sout_ref[2] = h2
            sout_ref[3] = c2

    return body


def kernel(x, w_ih_l0, w_hh_l0, b_l0, w_ih_l1, w_hh_l1, b_l1, w_fc, b_fc):
    B, T, I = x.shape
    H = w_hh_l0.shape[1]
    C = w_fc.shape[0]

    gate_scale = jnp.concatenate([
        jnp.full((2 * H,), 0.5, jnp.float32),
        jnp.ones((H,), jnp.float32),
        jnp.full((H,), 0.5, jnp.float32),
    ])[None, :]
    w1 = (jnp.concatenate([0.5 * w_hh_l0.T, w_ih_l0.T, b_l0[None, :]], axis=0)
          * gate_scale).astype(jnp.bfloat16)
    w2p = jnp.concatenate([0.5 * w_ih_l1.T, 0.5 * w_hh_l1.T], axis=0) * gate_scale
    b2p = b_l1[None, :] * gate_scale
    wfcp = jnp.zeros((H, 4 * H), jnp.float32).at[:H, :C].set(0.5 * w_fc.T)
    bfcp = jnp.zeros((1, 4 * H), jnp.float32).at[:, :C].set(b_fc[None, :])
    wpack = jnp.concatenate([w2p, b2p, wfcp, bfcp], axis=0)

    n_btiles = B // B_TILE
    const = lambda b: (0, 0)
    Tc = T // N_CHUNKS

    x_chunks = [
        jnp.transpose(x[:, k * Tc:(k + 1) * Tc, :].astype(jnp.bfloat16), (1, 0, 2))
        for k in range(N_CHUNKS)
    ]

    x_spec = pl.BlockSpec((Tc, B_TILE, I), lambda b: (0, b, 0))
    w1_spec = pl.BlockSpec((H + I + 1, 4 * H), const)
    wp_spec = pl.BlockSpec((386, 4 * H), const)
    state_spec = pl.BlockSpec((4, B_TILE, H), lambda b: (0, b, 0))
    cparams = pltpu.CompilerParams(dimension_semantics=("parallel",))

    res = None
    state = None
    for k in range(N_CHUNKS):
        first = k == 0
        last = k == N_CHUNKS - 1
        in_specs = [x_spec, w1_spec, wp_spec] + ([] if first else [state_spec])
        args = [x_chunks[k], w1, wpack] + ([] if first else [state])
        if last:
            out_shape = jax.ShapeDtypeStruct((B, C_PAD), jnp.float32)
            out_spec = pl.BlockSpec((B_TILE, C_PAD), lambda b: (b, 0))
        else:
            out_shape = jax.ShapeDtypeStruct((4, B, H), jnp.float32)
            out_spec = state_spec
        res = pl.pallas_call(
            _make_body(first, last),
            out_shape=out_shape,
            grid=(n_btiles,),
            in_specs=in_specs,
            out_specs=out_spec,
            compiler_params=cparams,
        )(*args)
        state = res

    return res[:, :C]
